# baseline (speedup 1.0000x reference)
import jax
import jax.numpy as jnp
from jax.experimental import pallas as pl
from jax.experimental.pallas import tpu as pltpu

INPUT_DIM = 64
ATTN_DIM = 128
SCALE = ATTN_DIM ** 0.5
LC = 8
CW = LC * INPUT_DIM


def _attn_kernel(tgt_ref, hist_ref, mask_ref, wq_ref, wk_ref, wv_ref, wo_ref,
                 out_ref, s_scr):
    tgt = tgt_ref[...]
    Bb = tgt.shape[0]
    nc = hist_ref.shape[1] // CW
    A = jnp.dot(wq_ref[...], wk_ref[...].T, preferred_element_type=jnp.float32)
    Wvo = jnp.dot(wv_ref[...], wo_ref[...], preferred_element_type=jnp.float32)
    qp = (jnp.dot(tgt, A, preferred_element_type=jnp.float32)
          * (1.0 / SCALE))
    qp_rep = pltpu.repeat(qp, LC, axis=1)

    xi = jax.lax.broadcasted_iota(jnp.int32, (CW, 128), 0) // INPUT_DIM
    ci = jax.lax.broadcasted_iota(jnp.int32, (CW, 128), 1)
    T = jnp.where(xi == ci, 1.0, 0.0).astype(jnp.bfloat16)
    ci2 = jax.lax.broadcasted_iota(jnp.int32, (128, CW), 0)
    xi2 = jax.lax.broadcasted_iota(jnp.int32, (128, CW), 1) // INPUT_DIM
    R = jnp.where(ci2 == xi2, 1.0, 0.0).astype(jnp.bfloat16)

    def score(i, _):
        hc = hist_ref[:, pl.ds(pl.multiple_of(i * CW, CW), CW)]
        p = (hc * qp_rep).astype(jnp.bfloat16)
        s8 = jnp.dot(p, T, preferred_element_type=jnp.float32)
        s_scr[pl.ds(i, 1), :, :] = s8[None]
        return 0

    jax.lax.fori_loop(0, nc, score, 0)

    def expand(i, acc):
        hc = hist_ref[:, pl.ds(pl.multiple_of(i * CW, CW), CW)]
        m8 = mask_ref[:, i, :, :].reshape(Bb, LC)
        m = pltpu.repeat(m8.astype(jnp.float32), 128 // LC, axis=1)
        s8 = s_scr[i, :, :]
        a8 = jnp.maximum(s8 * m, 0.0).astype(jnp.bfloat16)
        a_rep = jnp.dot(a8, R, preferred_element_type=jnp.float32)
        return acc + a_rep * hc

    acc = jax.lax.fori_loop(0, nc, expand, jnp.zeros((Bb, CW), jnp.float32))
    Wvo8 = jnp.broadcast_to(Wvo[None], (LC, INPUT_DIM, INPUT_DIM))
    Wvo8 = Wvo8.reshape(CW, INPUT_DIM)
    out_ref[...] = tgt + jnp.dot(acc, Wvo8,
                                 preferred_element_type=jnp.float32)


def kernel(target_item, history_sequence, mask, W_q, W_k, W_v, W_o):
    B, L, D = history_sequence.shape
    Bb = 128
    nc = L // LC
    hist2 = history_sequence.reshape(B, L * D)
    mask4 = mask.reshape(B, nc, 1, LC)
    grid = (B // Bb,)
    return pl.pallas_call(
        _attn_kernel,
        grid=grid,
        in_specs=[
            pl.BlockSpec((Bb, D), lambda i: (i, 0)),
            pl.BlockSpec((Bb, L * D), lambda i: (i, 0)),
            pl.BlockSpec((Bb, nc, 1, LC), lambda i: (i, 0, 0, 0)),
            pl.BlockSpec((D, ATTN_DIM), lambda i: (0, 0)),
            pl.BlockSpec((D, ATTN_DIM), lambda i: (0, 0)),
            pl.BlockSpec((D, ATTN_DIM), lambda i: (0, 0)),
            pl.BlockSpec((ATTN_DIM, D), lambda i: (0, 0)),
        ],
        out_specs=pl.BlockSpec((Bb, D), lambda i: (i, 0)),
        out_shape=jax.ShapeDtypeStruct((B, D), jnp.float32),
        scratch_shapes=[pltpu.VMEM((nc, Bb, 128), jnp.float32)],
        compiler_params=pltpu.CompilerParams(
            dimension_semantics=("arbitrary",),
        ),
    )(target_item, hist2, mask4, W_q, W_k, W_v, W_o)


# speedup vs baseline: 1.1629x; 1.1629x over previous
import jax
import jax.numpy as jnp
from jax.experimental import pallas as pl
from jax.experimental.pallas import tpu as pltpu

INPUT_DIM = 64
ATTN_DIM = 128
SCALE = ATTN_DIM ** 0.5
LC = 8
CW = LC * INPUT_DIM


def _attn_kernel(tgt_ref, hist_ref, mask_ref, wq_ref, wk_ref, wv_ref, wo_ref,
                 out_ref):
    tgt = tgt_ref[...]
    Bb = tgt.shape[0]
    nc = hist_ref.shape[1] // CW
    A = jnp.dot(wq_ref[...], wk_ref[...].T, preferred_element_type=jnp.float32)
    Wvo = jnp.dot(wv_ref[...], wo_ref[...], preferred_element_type=jnp.float32)
    qp = (jnp.dot(tgt, A, preferred_element_type=jnp.float32)
          * (1.0 / SCALE))
    qp_rep = pltpu.repeat(qp, LC, axis=1)

    xi = jax.lax.broadcasted_iota(jnp.int32, (CW, 128), 0) // INPUT_DIM
    ci = jax.lax.broadcasted_iota(jnp.int32, (CW, 128), 1)
    T = jnp.where(xi == ci, 1.0, 0.0).astype(jnp.bfloat16)
    ci2 = jax.lax.broadcasted_iota(jnp.int32, (128, CW), 0)
    xi2 = jax.lax.broadcasted_iota(jnp.int32, (128, CW), 1) // INPUT_DIM
    R = jnp.where(ci2 == xi2, 1.0, 0.0).astype(jnp.bfloat16)

    def score(i):
        hc = hist_ref[:, pl.ds(pl.multiple_of(i * CW, CW), CW)]
        p = (hc * qp_rep).astype(jnp.bfloat16)
        return jnp.dot(p, T, preferred_element_type=jnp.float32)

    def expand(i, s8, acc):
        hc = hist_ref[:, pl.ds(pl.multiple_of(i * CW, CW), CW)]
        m8 = mask_ref[:, i, :, :].reshape(Bb, LC)
        m = pltpu.repeat(m8.astype(jnp.float32), 128 // LC, axis=1)
        a8 = jnp.maximum(s8 * m, 0.0).astype(jnp.bfloat16)
        a_rep = jnp.dot(a8, R, preferred_element_type=jnp.float32)
        return acc + a_rep * hc

    def body(j, carry):
        acc, s8 = carry
        s8_next = score(j)
        acc = expand(j - 1, s8, acc)
        return acc, s8_next

    z = jnp.zeros((Bb, CW), jnp.float32)
    acc, s8_last = jax.lax.fori_loop(1, nc, body, (z, score(0)))
    acc = expand(nc - 1, s8_last, acc)
    Wvo8 = jnp.broadcast_to(Wvo[None], (LC, INPUT_DIM, INPUT_DIM))
    Wvo8 = Wvo8.reshape(CW, INPUT_DIM)
    out_ref[...] = tgt + jnp.dot(acc, Wvo8,
                                 preferred_element_type=jnp.float32)


def kernel(target_item, history_sequence, mask, W_q, W_k, W_v, W_o):
    B, L, D = history_sequence.shape
    Bb = 128
    nc = L // LC
    hist2 = history_sequence.reshape(B, L * D)
    mask4 = mask.reshape(B, nc, 1, LC)
    grid = (B // Bb,)
    return pl.pallas_call(
        _attn_kernel,
        grid=grid,
        in_specs=[
            pl.BlockSpec((Bb, D), lambda i: (i, 0)),
            pl.BlockSpec((Bb, L * D), lambda i: (i, 0)),
            pl.BlockSpec((Bb, nc, 1, LC), lambda i: (i, 0, 0, 0)),
            pl.BlockSpec((D, ATTN_DIM), lambda i: (0, 0)),
            pl.BlockSpec((D, ATTN_DIM), lambda i: (0, 0)),
            pl.BlockSpec((D, ATTN_DIM), lambda i: (0, 0)),
            pl.BlockSpec((ATTN_DIM, D), lambda i: (0, 0)),
        ],
        out_specs=pl.BlockSpec((Bb, D), lambda i: (i, 0)),
        out_shape=jax.ShapeDtypeStruct((B, D), jnp.float32),
        compiler_params=pltpu.CompilerParams(
            dimension_semantics=("arbitrary",),
        ),
    )(target_item, hist2, mask4, W_q, W_k, W_v, W_o)


# speedup vs baseline: 1.8499x; 1.5908x over previous
import jax
import jax.numpy as jnp
from jax.experimental import pallas as pl
from jax.experimental.pallas import tpu as pltpu

INPUT_DIM = 64
ATTN_DIM = 128
SCALE = ATTN_DIM ** 0.5
LC = 8
CW = LC * INPUT_DIM


def _attn_kernel(tgt_ref, hist_ref, mask_ref, wq_ref, wk_ref, wv_ref, wo_ref,
                 out_ref):
    tgt = tgt_ref[...]
    Bb = tgt.shape[0]
    nc = hist_ref.shape[1] // CW
    A = jnp.dot(wq_ref[...], wk_ref[...].T, preferred_element_type=jnp.float32)
    Wvo = jnp.dot(wv_ref[...], wo_ref[...], preferred_element_type=jnp.float32)
    qp = (jnp.dot(tgt, A, preferred_element_type=jnp.float32)
          * (1.0 / SCALE))
    qp_rep = pltpu.repeat(qp, LC, axis=1)

    xi = jax.lax.broadcasted_iota(jnp.int32, (CW, 128), 0) // INPUT_DIM
    ci = jax.lax.broadcasted_iota(jnp.int32, (CW, 128), 1)
    T = jnp.where(xi == ci, 1.0, 0.0).astype(jnp.bfloat16)
    ci2 = jax.lax.broadcasted_iota(jnp.int32, (128, CW), 0)
    xi2 = jax.lax.broadcasted_iota(jnp.int32, (128, CW), 1) // INPUT_DIM
    R = jnp.where(ci2 == xi2, 1.0, 0.0).astype(jnp.bfloat16)

    def score(i):
        hc = hist_ref[:, pl.ds(pl.multiple_of(i * CW, CW), CW)]
        p = (hc * qp_rep).astype(jnp.bfloat16)
        return jnp.dot(p, T, preferred_element_type=jnp.float32)

    def expand(i, s8, acc):
        hc = hist_ref[:, pl.ds(pl.multiple_of(i * CW, CW), CW)]
        m = mask_ref[:, pl.ds(pl.multiple_of(i * 128, 128), 128)]
        a8 = jnp.maximum(s8 * m.astype(jnp.float32),
                         0.0).astype(jnp.bfloat16)
        a_rep = jnp.dot(a8, R, preferred_element_type=jnp.float32)
        return acc + a_rep * hc

    def body(j, carry):
        acc, s8 = carry
        s8_next = score(j)
        acc = expand(j - 1, s8, acc)
        return acc, s8_next

    z = jnp.zeros((Bb, CW), jnp.float32)
    acc, s8_last = jax.lax.fori_loop(1, nc, body, (z, score(0)))
    acc = expand(nc - 1, s8_last, acc)
    Wvo8 = jnp.broadcast_to(Wvo[None], (LC, INPUT_DIM, INPUT_DIM))
    Wvo8 = Wvo8.reshape(CW, INPUT_DIM)
    out_ref[...] = tgt + jnp.dot(acc, Wvo8,
                                 preferred_element_type=jnp.float32)


def kernel(target_item, history_sequence, mask, W_q, W_k, W_v, W_o):
    B, L, D = history_sequence.shape
    Bb = 128
    nc = L // LC
    hist2 = history_sequence.reshape(B, L * D)
    mask_exp = jnp.broadcast_to(
        mask.reshape(B, nc, 1, LC).astype(jnp.bfloat16),
        (B, nc, 128 // LC, LC)).reshape(B, nc * 128)
    grid = (B // Bb,)
    return pl.pallas_call(
        _attn_kernel,
        grid=grid,
        in_specs=[
            pl.BlockSpec((Bb, D), lambda i: (i, 0)),
            pl.BlockSpec((Bb, L * D), lambda i: (i, 0)),
            pl.BlockSpec((Bb, nc * 128), lambda i: (i, 0)),
            pl.BlockSpec((D, ATTN_DIM), lambda i: (0, 0)),
            pl.BlockSpec((D, ATTN_DIM), lambda i: (0, 0)),
            pl.BlockSpec((D, ATTN_DIM), lambda i: (0, 0)),
            pl.BlockSpec((ATTN_DIM, D), lambda i: (0, 0)),
        ],
        out_specs=pl.BlockSpec((Bb, D), lambda i: (i, 0)),
        out_shape=jax.ShapeDtypeStruct((B, D), jnp.float32),
        compiler_params=pltpu.CompilerParams(
            dimension_semantics=("arbitrary",),
        ),
    )(target_item, hist2, mask_exp, W_q, W_k, W_v, W_o)


# speedup vs baseline: 2.0831x; 1.1261x over previous
import jax
import jax.numpy as jnp
from jax.experimental import pallas as pl
from jax.experimental.pallas import tpu as pltpu

INPUT_DIM = 64
ATTN_DIM = 128
SCALE = ATTN_DIM ** 0.5
LC = 8
CW = LC * INPUT_DIM


def _attn_kernel(tgt_ref, hist_ref, mask_ref, wq_ref, wk_ref, wv_ref, wo_ref,
                 out_ref):
    tgt = tgt_ref[...]
    Bb = tgt.shape[0]
    nc = hist_ref.shape[1] // CW
    A = jnp.dot(wq_ref[...], wk_ref[...].T, preferred_element_type=jnp.float32)
    Wvo = jnp.dot(wv_ref[...], wo_ref[...], preferred_element_type=jnp.float32)
    qp = (jnp.dot(tgt, A, preferred_element_type=jnp.float32)
          * (1.0 / SCALE))
    qp_rep = pltpu.repeat(qp, LC, axis=1)

    xi = jax.lax.broadcasted_iota(jnp.int32, (CW, 128), 0) // INPUT_DIM
    ci = jax.lax.broadcasted_iota(jnp.int32, (CW, 128), 1)
    T = jnp.where(xi == ci, 1.0, 0.0).astype(jnp.bfloat16)
    ci2 = jax.lax.broadcasted_iota(jnp.int32, (128, CW), 0)
    xi2 = jax.lax.broadcasted_iota(jnp.int32, (128, CW), 1) // INPUT_DIM
    R = jnp.where(ci2 == xi2, 1.0, 0.0).astype(jnp.bfloat16)

    def score(i):
        hc = hist_ref[:, pl.ds(pl.multiple_of(i * CW, CW), CW)]
        p = (hc * qp_rep).astype(jnp.bfloat16)
        return jnp.dot(p, T, preferred_element_type=jnp.float32)

    def expand(i, s8, acc):
        hc = hist_ref[:, pl.ds(pl.multiple_of(i * CW, CW), CW)]
        m = mask_ref[:, pl.ds(pl.multiple_of(i * 128, 128), 128)]
        a8 = jnp.maximum(s8 * m.astype(jnp.float32),
                         0.0).astype(jnp.bfloat16)
        a_rep = jnp.dot(a8, R, preferred_element_type=jnp.float32)
        return acc + a_rep * hc

    def body(j, carry):
        acc0, acc1, s8a, s8b = carry
        sa_next = score(2 * j)
        sb_next = score(2 * j + 1)
        acc0 = expand(2 * j - 2, s8a, acc0)
        acc1 = expand(2 * j - 1, s8b, acc1)
        return acc0, acc1, sa_next, sb_next

    z = jnp.zeros((Bb, CW), jnp.float32)
    acc0, acc1, s8a, s8b = jax.lax.fori_loop(
        1, nc // 2, body, (z, z, score(0), score(1)))
    acc0 = expand(nc - 3, s8a, acc0)
    acc1 = expand(nc - 2, s8b, acc1)
    acc = expand(nc - 1, score(nc - 1), acc0 + acc1)
    Wvo8 = jnp.broadcast_to(Wvo[None], (LC, INPUT_DIM, INPUT_DIM))
    Wvo8 = Wvo8.reshape(CW, INPUT_DIM)
    out_ref[...] = tgt + jnp.dot(acc, Wvo8,
                                 preferred_element_type=jnp.float32)


def kernel(target_item, history_sequence, mask, W_q, W_k, W_v, W_o):
    B, L, D = history_sequence.shape
    Bb = 128
    nc = L // LC
    hist2 = history_sequence.reshape(B, L * D)
    mask_exp = jnp.broadcast_to(
        mask.reshape(B, nc, 1, LC).astype(jnp.bfloat16),
        (B, nc, 128 // LC, LC)).reshape(B, nc * 128)
    grid = (B // Bb,)
    return pl.pallas_call(
        _attn_kernel,
        grid=grid,
        in_specs=[
            pl.BlockSpec((Bb, D), lambda i: (i, 0)),
            pl.BlockSpec((Bb, L * D), lambda i: (i, 0)),
            pl.BlockSpec((Bb, nc * 128), lambda i: (i, 0)),
            pl.BlockSpec((D, ATTN_DIM), lambda i: (0, 0)),
            pl.BlockSpec((D, ATTN_DIM), lambda i: (0, 0)),
            pl.BlockSpec((D, ATTN_DIM), lambda i: (0, 0)),
            pl.BlockSpec((ATTN_DIM, D), lambda i: (0, 0)),
        ],
        out_specs=pl.BlockSpec((Bb, D), lambda i: (i, 0)),
        out_shape=jax.ShapeDtypeStruct((B, D), jnp.float32),
        compiler_params=pltpu.CompilerParams(
            dimension_semantics=("arbitrary",),
        ),
    )(target_item, hist2, mask_exp, W_q, W_k, W_v, W_o)


# speedup vs baseline: 2.2608x; 1.0853x over previous
import jax
import jax.numpy as jnp
from jax.experimental import pallas as pl
from jax.experimental.pallas import tpu as pltpu

INPUT_DIM = 64
ATTN_DIM = 128
SCALE = ATTN_DIM ** 0.5
LC = 8
CW = LC * INPUT_DIM


def _attn_kernel(tgt_ref, hist_ref, mask_ref, wq_ref, wk_ref, wv_ref, wo_ref,
                 out_ref):
    tgt = tgt_ref[...]
    Bb = tgt.shape[0]
    nc = hist_ref.shape[1] // CW
    A = jnp.dot(wq_ref[...], wk_ref[...].T, preferred_element_type=jnp.float32)
    Wvo = jnp.dot(wv_ref[...], wo_ref[...], preferred_element_type=jnp.float32)
    qp = (jnp.dot(tgt, A, preferred_element_type=jnp.float32)
          * (1.0 / SCALE))
    qp_rep = pltpu.repeat(qp, LC, axis=1)

    xi = jax.lax.broadcasted_iota(jnp.int32, (CW, 128), 0) // INPUT_DIM
    ci = jax.lax.broadcasted_iota(jnp.int32, (CW, 128), 1)
    T = jnp.where(xi == ci, 1.0, 0.0).astype(jnp.bfloat16)
    ci2 = jax.lax.broadcasted_iota(jnp.int32, (128, CW), 0)
    xi2 = jax.lax.broadcasted_iota(jnp.int32, (128, CW), 1) // INPUT_DIM
    R = jnp.where(ci2 == xi2, 1.0, 0.0).astype(jnp.bfloat16)

    def score(i):
        hc = hist_ref[:, pl.ds(pl.multiple_of(i * CW, CW), CW)]
        p = (hc * qp_rep).astype(jnp.bfloat16)
        return jnp.dot(p, T, preferred_element_type=jnp.float32)

    def expand(i, s8, acc):
        hc = hist_ref[:, pl.ds(pl.multiple_of(i * CW, CW), CW)]
        m = mask_ref[:, pl.ds(pl.multiple_of(i * 128, 128), 128)]
        a8 = jnp.maximum(s8 * m.astype(jnp.float32),
                         0.0).astype(jnp.bfloat16)
        a_rep = jnp.dot(a8, R, preferred_element_type=jnp.float32)
        return acc + a_rep * hc

    def body(j, carry):
        acc0, acc1, s8a, s8b = carry
        sa_next = score(2 * j)
        sb_next = score(2 * j + 1)
        acc0 = expand(2 * j - 2, s8a, acc0)
        acc1 = expand(2 * j - 1, s8b, acc1)
        return acc0, acc1, sa_next, sb_next

    z = jnp.zeros((Bb, CW), jnp.float32)
    acc0, acc1, s8a, s8b = jax.lax.fori_loop(
        1, nc // 2, body, (z, z, score(0), score(1)))
    acc0 = expand(nc - 3, s8a, acc0)
    acc1 = expand(nc - 2, s8b, acc1)
    acc = expand(nc - 1, score(nc - 1), acc0 + acc1)
    Wvo8 = jnp.broadcast_to(Wvo[None], (LC, INPUT_DIM, INPUT_DIM))
    Wvo8 = Wvo8.reshape(CW, INPUT_DIM)
    out_ref[...] = tgt + jnp.dot(acc, Wvo8,
                                 preferred_element_type=jnp.float32)


def kernel(target_item, history_sequence, mask, W_q, W_k, W_v, W_o):
    B, L, D = history_sequence.shape
    Bb = 256
    nc = L // LC
    hist2 = history_sequence.reshape(B, L * D)
    mask_exp = jnp.broadcast_to(
        mask.reshape(B, nc, 1, LC).astype(jnp.bfloat16),
        (B, nc, 128 // LC, LC)).reshape(B, nc * 128)
    grid = (B // Bb,)
    return pl.pallas_call(
        _attn_kernel,
        grid=grid,
        in_specs=[
            pl.BlockSpec((Bb, D), lambda i: (i, 0)),
            pl.BlockSpec((Bb, L * D), lambda i: (i, 0)),
            pl.BlockSpec((Bb, nc * 128), lambda i: (i, 0)),
            pl.BlockSpec((D, ATTN_DIM), lambda i: (0, 0)),
            pl.BlockSpec((D, ATTN_DIM), lambda i: (0, 0)),
            pl.BlockSpec((D, ATTN_DIM), lambda i: (0, 0)),
            pl.BlockSpec((ATTN_DIM, D), lambda i: (0, 0)),
        ],
        out_specs=pl.BlockSpec((Bb, D), lambda i: (i, 0)),
        out_shape=jax.ShapeDtypeStruct((B, D), jnp.float32),
        compiler_params=pltpu.CompilerParams(
            dimension_semantics=("arbitrary",),
        ),
    )(target_item, hist2, mask_exp, W_q, W_k, W_v, W_o)
